# baseline (speedup 1.0000x reference)
import functools

import jax
import jax.numpy as jnp
from jax import lax
from jax.experimental import pallas as pl
from jax.experimental.pallas import tpu as pltpu
from jax.experimental.pallas import tpu_sc as plsc

D_MODEL = 1024
HIDDEN = 4096
NUM_EXPERTS = 8
SEQ = 2048
A_TOTAL = 2 * SEQ

R_TILE = 256
T_MAX = A_TOTAL // R_TILE + NUM_EXPERTS - 1
N_PAD = T_MAX * R_TILE
H_CHUNK = 1024
N_H = HIDDEN // H_CHUNK
S_STEPS = T_MAX * N_H
NW = 32
TPW = SEQ // NW


def _shift_down(x, d, axis):
    pad = jnp.zeros_like(lax.slice_in_dim(x, 0, d, axis=axis))
    rest = lax.slice_in_dim(x, 0, x.shape[axis] - d, axis=axis)
    return lax.concatenate([pad, rest], dimension=axis)


def _lane_cumsum(m):
    acc = m
    d = 1
    while d < m.shape[1]:
        acc = acc + _shift_down(acc, d, 1)
        d *= 2
    return acc


def _router_kernel(xT_ref, gwT_ref, gb_ref, p0_ref, p1_ref, wA_ref, wB_ref,
                   sched_ref):
    logits = jnp.dot(gwT_ref[...], xT_ref[...],
                     preferred_element_type=jnp.float32) + gb_ref[...]
    m = jnp.max(logits, axis=0, keepdims=True)
    ex = jnp.exp(logits - m)
    probs = ex / jnp.sum(ex, axis=0, keepdims=True)

    row = lax.broadcasted_iota(jnp.int32, probs.shape, 0)
    m1 = jnp.max(probs, axis=0, keepdims=True)
    i1 = jnp.min(jnp.where(probs == m1, row, NUM_EXPERTS), axis=0,
                 keepdims=True)
    mask1 = row == i1
    probs2 = jnp.where(mask1, -1.0, probs)
    m2 = jnp.max(probs2, axis=0, keepdims=True)
    i2 = jnp.min(jnp.where(probs2 == m2, row, NUM_EXPERTS), axis=0,
                 keepdims=True)
    mask2 = row == i2

    M0 = mask1.astype(jnp.float32)
    M1 = mask2.astype(jnp.float32)
    C0 = _lane_cumsum(M0)
    C1 = _lane_cumsum(M1)
    c0 = C0[:, SEQ - 1:SEQ]
    c1 = C1[:, SEQ - 1:SEQ]
    cnt = (c0 + c1).astype(jnp.int32)

    nt = (cnt + (R_TILE - 1)) // R_TILE
    row81 = lax.broadcasted_iota(jnp.int32, (NUM_EXPERTS, 1), 0)
    acc = nt
    d = 1
    while d < NUM_EXPERTS:
        acc = acc + _shift_down(acc, d, 0)
        d *= 2
    tilebase = acc - nt
    offp = (tilebase * R_TILE).astype(jnp.float32)

    pos0 = jnp.sum(M0 * (offp + C0 - 1.0), axis=0, keepdims=True)
    pos1 = jnp.sum(M1 * (offp + c0 + C1 - 1.0), axis=0, keepdims=True)
    p0_ref[...] = pos0.astype(jnp.int32)
    p1_ref[...] = pos1.astype(jnp.int32)
    wA_ref[...] = m1
    wB_ref[...] = m2

    tb_f = tilebase.astype(jnp.float32)
    nt_f = nt.astype(jnp.float32)
    s_f = lax.broadcasted_iota(jnp.int32, (1, 128), 1).astype(jnp.float32)
    z = jnp.zeros((1, 128), jnp.float32)
    a_tile, a_hc, a_e, a_v = z, z, z, z
    for e in range(NUM_EXPERTS):
        sel = (row81 == e).astype(jnp.float32)
        tb_e = jnp.sum(sel * tb_f)
        nt_e = jnp.sum(sel * nt_f)
        sb_e = 4.0 * tb_e
        in_e = jnp.logical_and(s_f >= sb_e, s_f < sb_e + 4.0 * nt_e)
        ds = s_f - sb_e
        q = jnp.floor((ds + 0.5) / nt_e)
        rem = ds - q * nt_e
        a_tile = a_tile + jnp.where(in_e, tb_e + rem, 0.0)
        a_hc = a_hc + jnp.where(in_e, q, 0.0)
        a_e = a_e + jnp.where(in_e, float(e), 0.0)
        a_v = a_v + jnp.where(in_e, 1.0, 0.0)
    sched_ref[0:1, :] = a_tile.astype(jnp.int32)
    sched_ref[1:2, :] = a_hc.astype(jnp.int32)
    sched_ref[2:3, :] = a_e.astype(jnp.int32)
    sched_ref[3:4, :] = a_v.astype(jnp.int32)


def _router(xT, gwT, gb2):
    return pl.pallas_call(
        _router_kernel,
        out_shape=(
            jax.ShapeDtypeStruct((1, SEQ), jnp.int32),
            jax.ShapeDtypeStruct((1, SEQ), jnp.int32),
            jax.ShapeDtypeStruct((1, SEQ), jnp.float32),
            jax.ShapeDtypeStruct((1, SEQ), jnp.float32),
            jax.ShapeDtypeStruct((4, 128), jnp.int32),
        ),
    )(xT, gwT, gb2)


@functools.cache
def _get_dispatch():
    return functools.partial(
        pl.kernel,
        out_type=(
            jax.ShapeDtypeStruct((N_PAD, D_MODEL // 2), jnp.int32),
            jax.ShapeDtypeStruct((N_PAD, 128), jnp.float32),
        ),
        mesh=plsc.VectorSubcoreMesh(core_axis_name="c", subcore_axis_name="s"),
        scratch_types=[
            pltpu.VMEM((TPW, D_MODEL // 2), jnp.int32),
            pltpu.VMEM((TPW, 128), jnp.float32),
            pltpu.VMEM((TPW, 128), jnp.float32),
            pltpu.VMEM((TPW,), jnp.int32),
            pltpu.VMEM((TPW,), jnp.int32),
            pltpu.SemaphoreType.DMA,
        ],
    )(_dispatch_body)


def _dispatch_body(x_hbm, wf_hbm, posw_hbm, xs_hbm, ws_hbm,
                   xrows, wr0, wr1, pidx0, pidx1, sem):
    wid = lax.axis_index("s") * 2 + lax.axis_index("c")
    base = wid * TPW
    pltpu.sync_copy(posw_hbm.at[wid, 0], pidx0)
    pltpu.sync_copy(posw_hbm.at[wid, 1], pidx1)
    pltpu.sync_copy(x_hbm.at[pl.ds(base, TPW)], xrows)
    pltpu.sync_copy(wf_hbm.at[pl.ds(base, TPW)], wr0)
    pltpu.sync_copy(wf_hbm.at[pl.ds(SEQ + base, TPW)], wr1)
    pltpu.async_copy(xrows, xs_hbm.at[pidx0], sem).wait()
    pltpu.async_copy(xrows, xs_hbm.at[pidx1], sem).wait()
    pltpu.async_copy(wr0, ws_hbm.at[pidx0], sem).wait()
    pltpu.async_copy(wr1, ws_hbm.at[pidx1], sem).wait()


def _gmm_kernel(st, sh, se, sv, xs_ref, ws_ref, wi_ref, bi_ref, wo_ref,
                bo_ref, y_ref, wi_bf, wo_bf):
    s = pl.program_id(0)
    hc = sh[s]
    rows = pl.ds(st[s] * R_TILE, R_TILE)
    prev = jnp.maximum(s - 1, 0)
    new_w = jnp.logical_or(
        s == 0, jnp.logical_or(se[s] != se[prev], sh[s] != sh[prev]))
    valid = sv[s] == 1

    @pl.when(jnp.logical_and(valid, new_w))
    def _():
        wi_bf[...] = wi_ref[0].astype(jnp.bfloat16)
        wo_bf[...] = wo_ref[0].astype(jnp.bfloat16)

    @pl.when(valid)
    def _():
        xb = xs_ref[rows, :]
        w_col = ws_ref[...][:, 0:1]
        h = jnp.dot(xb, wi_bf[...], preferred_element_type=jnp.float32)
        h = jnp.maximum(h + bi_ref[0], 0.0)
        part = jnp.dot(h.astype(jnp.bfloat16), wo_bf[...],
                       preferred_element_type=jnp.float32)

        @pl.when(hc == 0)
        def _():
            y_ref[rows, :] = (part + bo_ref[0]) * w_col

        @pl.when(hc != 0)
        def _():
            y_ref[rows, :] += part * w_col


def _gmm(st, sh, se, sv, xs, ws, Wi, bi3, Wo, bo3):
    grid_spec = pltpu.PrefetchScalarGridSpec(
        num_scalar_prefetch=4,
        grid=(S_STEPS,),
        in_specs=[
            pl.BlockSpec((N_PAD, D_MODEL),
                         lambda s, st, sh, se, sv: (0, 0)),
            pl.BlockSpec((R_TILE, 128),
                         lambda s, st, sh, se, sv: (st[s], 0)),
            pl.BlockSpec((1, D_MODEL, H_CHUNK),
                         lambda s, st, sh, se, sv: (se[s], 0, sh[s])),
            pl.BlockSpec((1, 1, H_CHUNK),
                         lambda s, st, sh, se, sv: (se[s], 0, sh[s])),
            pl.BlockSpec((1, H_CHUNK, D_MODEL),
                         lambda s, st, sh, se, sv: (se[s], sh[s], 0)),
            pl.BlockSpec((1, 1, D_MODEL),
                         lambda s, st, sh, se, sv: (se[s], 0, 0)),
        ],
        out_specs=pl.BlockSpec((N_PAD, D_MODEL),
                               lambda s, st, sh, se, sv: (0, 0)),
        scratch_shapes=[
            pltpu.VMEM((D_MODEL, H_CHUNK), jnp.bfloat16),
            pltpu.VMEM((H_CHUNK, D_MODEL), jnp.bfloat16),
        ],
    )
    return pl.pallas_call(
        _gmm_kernel,
        grid_spec=grid_spec,
        out_shape=jax.ShapeDtypeStruct((N_PAD, D_MODEL), jnp.float32),
    )(st, sh, se, sv, xs, ws, Wi, bi3, Wo, bo3)


CH = 32

@functools.cache
def _get_combine():
    return functools.partial(
        pl.kernel,
        out_type=jax.ShapeDtypeStruct((SEQ, D_MODEL), jnp.float32),
        mesh=plsc.VectorSubcoreMesh(core_axis_name="c", subcore_axis_name="s"),
        scratch_types=[
            pltpu.VMEM((CH, D_MODEL), jnp.float32),
            pltpu.VMEM((CH, D_MODEL), jnp.float32),
            pltpu.VMEM((CH,), jnp.int32),
            pltpu.VMEM((CH,), jnp.int32),
            pltpu.SemaphoreType.DMA,
            pltpu.SemaphoreType.DMA,
        ],
    )(_combine_body)


def _combine_body(y_hbm, posw_hbm, out_hbm, r0, r1, i0, i1, sem0, sem1):
    wid = lax.axis_index("s") * 2 + lax.axis_index("c")
    base = wid * TPW
    for c in range(TPW // CH):
        tb = base + c * CH
        pltpu.sync_copy(posw_hbm.at[wid, 0, pl.ds(c * CH, CH)], i0)
        pltpu.sync_copy(posw_hbm.at[wid, 1, pl.ds(c * CH, CH)], i1)
        cp0 = pltpu.async_copy(y_hbm.at[i0], r0, sem0)
        cp1 = pltpu.async_copy(y_hbm.at[i1], r1, sem1)
        cp0.wait()
        cp1.wait()

        def body(j, _):
            for u in range(D_MODEL // 16):
                sl = pl.ds(u * 16, 16)
                r0[j, sl] = r0[j, sl] + r1[j, sl]
            return 0

        lax.fori_loop(0, CH, body, 0)
        pltpu.sync_copy(r0, out_hbm.at[pl.ds(tb, CH)])


@jax.jit
def kernel(x, gate_W, gate_b, Wi, bi, Wo, bo):
    b, s, d = x.shape
    x2 = x.reshape(s, d)
    xT = x2.T
    gwT = gate_W.T
    gb2 = gate_b.reshape(NUM_EXPERTS, 1)

    p0, p1, wA, wB, sched = _router(xT, gwT, gb2)
    st, sh, se, sv = sched[0], sched[1], sched[2], sched[3]

    posw = jnp.stack([p0.reshape(NW, TPW), p1.reshape(NW, TPW)], axis=1)
    wf16 = jnp.broadcast_to(
        jnp.concatenate([wA, wB], axis=1).reshape(A_TOTAL, 1), (A_TOTAL, 128))

    xi = lax.bitcast_convert_type(
        x2.astype(jnp.bfloat16).reshape(SEQ, D_MODEL // 2, 2), jnp.int32)
    xs_i, ws = _get_dispatch()(xi, wf16, posw)
    xs = lax.bitcast_convert_type(xs_i, jnp.bfloat16).reshape(N_PAD, D_MODEL)
    y = _gmm(st, sh, se, sv, xs, ws, Wi,
             bi.reshape(NUM_EXPERTS, 1, HIDDEN), Wo,
             bo.reshape(NUM_EXPERTS, 1, D_MODEL))
    out = _get_combine()(y, posw)
    return out.reshape(b, s, d)


# speedup vs baseline: 1.5542x; 1.5542x over previous
import functools

import jax
import jax.numpy as jnp
from jax import lax
from jax.experimental import pallas as pl
from jax.experimental.pallas import tpu as pltpu
from jax.experimental.pallas import tpu_sc as plsc

D_MODEL = 1024
HIDDEN = 4096
NUM_EXPERTS = 8
SEQ = 2048
A_TOTAL = 2 * SEQ

R_TILE = 256
T_MAX = A_TOTAL // R_TILE + NUM_EXPERTS - 1
N_PAD = T_MAX * R_TILE
H_CHUNK = 1024
N_H = HIDDEN // H_CHUNK
S_STEPS = T_MAX * N_H
NW = 32
TPW = SEQ // NW


def _shift_down(x, d, axis):
    pad = jnp.zeros_like(lax.slice_in_dim(x, 0, d, axis=axis))
    rest = lax.slice_in_dim(x, 0, x.shape[axis] - d, axis=axis)
    return lax.concatenate([pad, rest], dimension=axis)


def _lane_cumsum(m):
    acc = m
    d = 1
    while d < m.shape[1]:
        acc = acc + _shift_down(acc, d, 1)
        d *= 2
    return acc


def _router_kernel(xT_ref, gwT_ref, gb_ref, p0_ref, p1_ref, wA_ref, wB_ref,
                   sched_ref):
    logits = jnp.dot(gwT_ref[...], xT_ref[...],
                     preferred_element_type=jnp.float32) + gb_ref[...]
    m = jnp.max(logits, axis=0, keepdims=True)
    ex = jnp.exp(logits - m)
    probs = ex / jnp.sum(ex, axis=0, keepdims=True)

    row = lax.broadcasted_iota(jnp.int32, probs.shape, 0)
    m1 = jnp.max(probs, axis=0, keepdims=True)
    i1 = jnp.min(jnp.where(probs == m1, row, NUM_EXPERTS), axis=0,
                 keepdims=True)
    mask1 = row == i1
    probs2 = jnp.where(mask1, -1.0, probs)
    m2 = jnp.max(probs2, axis=0, keepdims=True)
    i2 = jnp.min(jnp.where(probs2 == m2, row, NUM_EXPERTS), axis=0,
                 keepdims=True)
    mask2 = row == i2

    M0 = mask1.astype(jnp.float32)
    M1 = mask2.astype(jnp.float32)
    C0 = _lane_cumsum(M0)
    C1 = _lane_cumsum(M1)
    c0 = C0[:, SEQ - 1:SEQ]
    c1 = C1[:, SEQ - 1:SEQ]
    cnt = (c0 + c1).astype(jnp.int32)

    nt = (cnt + (R_TILE - 1)) // R_TILE
    row81 = lax.broadcasted_iota(jnp.int32, (NUM_EXPERTS, 1), 0)
    acc = nt
    d = 1
    while d < NUM_EXPERTS:
        acc = acc + _shift_down(acc, d, 0)
        d *= 2
    tilebase = acc - nt
    offp = (tilebase * R_TILE).astype(jnp.float32)

    pos0 = jnp.sum(M0 * (offp + C0 - 1.0), axis=0, keepdims=True)
    pos1 = jnp.sum(M1 * (offp + c0 + C1 - 1.0), axis=0, keepdims=True)
    p0_ref[...] = pos0.astype(jnp.int32)
    p1_ref[...] = pos1.astype(jnp.int32)
    wA_ref[...] = m1
    wB_ref[...] = m2

    tb_f = tilebase.astype(jnp.float32)
    nt_f = nt.astype(jnp.float32)
    s_f = lax.broadcasted_iota(jnp.int32, (1, 128), 1).astype(jnp.float32)
    z = jnp.zeros((1, 128), jnp.float32)
    a_tile, a_hc, a_e, a_v = z, z, z, z
    for e in range(NUM_EXPERTS):
        sel = (row81 == e).astype(jnp.float32)
        tb_e = jnp.sum(sel * tb_f)
        nt_e = jnp.sum(sel * nt_f)
        sb_e = 4.0 * tb_e
        in_e = jnp.logical_and(s_f >= sb_e, s_f < sb_e + 4.0 * nt_e)
        ds = s_f - sb_e
        q = jnp.floor((ds + 0.5) / nt_e)
        rem = ds - q * nt_e
        a_tile = a_tile + jnp.where(in_e, tb_e + rem, 0.0)
        a_hc = a_hc + jnp.where(in_e, q, 0.0)
        a_e = a_e + jnp.where(in_e, float(e), 0.0)
        a_v = a_v + jnp.where(in_e, 1.0, 0.0)
    sched_ref[0:1, :] = a_tile.astype(jnp.int32)
    sched_ref[1:2, :] = a_hc.astype(jnp.int32)
    sched_ref[2:3, :] = a_e.astype(jnp.int32)
    sched_ref[3:4, :] = a_v.astype(jnp.int32)


def _router(xT, gwT, gb2):
    return pl.pallas_call(
        _router_kernel,
        out_shape=(
            jax.ShapeDtypeStruct((1, SEQ), jnp.int32),
            jax.ShapeDtypeStruct((1, SEQ), jnp.int32),
            jax.ShapeDtypeStruct((1, SEQ), jnp.float32),
            jax.ShapeDtypeStruct((1, SEQ), jnp.float32),
            jax.ShapeDtypeStruct((4, 128), jnp.int32),
        ),
    )(xT, gwT, gb2)


@functools.cache
def _get_dispatch():
    return functools.partial(
        pl.kernel,
        out_type=(
            jax.ShapeDtypeStruct((N_PAD, D_MODEL), jnp.float32),
            jax.ShapeDtypeStruct((N_PAD, 128), jnp.float32),
        ),
        mesh=plsc.VectorSubcoreMesh(core_axis_name="c", subcore_axis_name="s"),
        scratch_types=[
            pltpu.VMEM((TPW, D_MODEL), jnp.float32),
            pltpu.VMEM((TPW, 128), jnp.float32),
            pltpu.VMEM((TPW, 128), jnp.float32),
            pltpu.VMEM((TPW,), jnp.int32),
            pltpu.VMEM((TPW,), jnp.int32),
            pltpu.SemaphoreType.DMA,
        ],
    )(_dispatch_body)


def _dispatch_body(x_hbm, wf_hbm, posw_hbm, xs_hbm, ws_hbm,
                   xrows, wr0, wr1, pidx0, pidx1, sem):
    wid = lax.axis_index("s") * 2 + lax.axis_index("c")
    base = wid * TPW
    pltpu.sync_copy(posw_hbm.at[wid, 0], pidx0)
    pltpu.sync_copy(posw_hbm.at[wid, 1], pidx1)
    pltpu.sync_copy(x_hbm.at[pl.ds(base, TPW)], xrows)
    pltpu.sync_copy(wf_hbm.at[pl.ds(base, TPW)], wr0)
    pltpu.sync_copy(wf_hbm.at[pl.ds(SEQ + base, TPW)], wr1)
    pltpu.async_copy(xrows, xs_hbm.at[pidx0], sem).wait()
    pltpu.async_copy(xrows, xs_hbm.at[pidx1], sem).wait()
    pltpu.async_copy(wr0, ws_hbm.at[pidx0], sem).wait()
    pltpu.async_copy(wr1, ws_hbm.at[pidx1], sem).wait()


def _gmm_kernel(st, sh, se, sv, xs_ref, ws_ref, wi_ref, bi_ref, wo_ref,
                bo_ref, y_ref, wi_bf, wo_bf):
    s = pl.program_id(0)
    hc = sh[s]
    rows = pl.ds(st[s] * R_TILE, R_TILE)
    prev = jnp.maximum(s - 1, 0)
    new_w = jnp.logical_or(
        s == 0, jnp.logical_or(se[s] != se[prev], sh[s] != sh[prev]))
    valid = sv[s] == 1

    @pl.when(jnp.logical_and(valid, new_w))
    def _():
        wi_bf[...] = wi_ref[0].astype(jnp.bfloat16)
        wo_bf[...] = wo_ref[0].astype(jnp.bfloat16)

    @pl.when(valid)
    def _():
        xb = xs_ref[...].astype(jnp.bfloat16)
        w_col = ws_ref[...][:, 0:1]
        h = jnp.dot(xb, wi_bf[...], preferred_element_type=jnp.float32)
        h = jnp.maximum(h + bi_ref[0], 0.0)
        part = jnp.dot(h.astype(jnp.bfloat16), wo_bf[...],
                       preferred_element_type=jnp.float32)

        @pl.when(hc == 0)
        def _():
            y_ref[rows, :] = (part + bo_ref[0]) * w_col

        @pl.when(hc != 0)
        def _():
            y_ref[rows, :] += part * w_col


def _gmm(st, sh, se, sv, xs, ws, Wi, bi3, Wo, bo3):
    grid_spec = pltpu.PrefetchScalarGridSpec(
        num_scalar_prefetch=4,
        grid=(S_STEPS,),
        in_specs=[
            pl.BlockSpec((R_TILE, D_MODEL),
                         lambda s, st, sh, se, sv: (st[s], 0)),
            pl.BlockSpec((R_TILE, 128),
                         lambda s, st, sh, se, sv: (st[s], 0)),
            pl.BlockSpec((1, D_MODEL, H_CHUNK),
                         lambda s, st, sh, se, sv: (se[s], 0, sh[s])),
            pl.BlockSpec((1, 1, H_CHUNK),
                         lambda s, st, sh, se, sv: (se[s], 0, sh[s])),
            pl.BlockSpec((1, H_CHUNK, D_MODEL),
                         lambda s, st, sh, se, sv: (se[s], sh[s], 0)),
            pl.BlockSpec((1, 1, D_MODEL),
                         lambda s, st, sh, se, sv: (se[s], 0, 0)),
        ],
        out_specs=pl.BlockSpec((N_PAD, D_MODEL),
                               lambda s, st, sh, se, sv: (0, 0)),
        scratch_shapes=[
            pltpu.VMEM((D_MODEL, H_CHUNK), jnp.bfloat16),
            pltpu.VMEM((H_CHUNK, D_MODEL), jnp.bfloat16),
        ],
    )
    return pl.pallas_call(
        _gmm_kernel,
        grid_spec=grid_spec,
        out_shape=jax.ShapeDtypeStruct((N_PAD, D_MODEL), jnp.float32),
    )(st, sh, se, sv, xs, ws, Wi, bi3, Wo, bo3)


CH = 32

@functools.cache
def _get_combine():
    return functools.partial(
        pl.kernel,
        out_type=jax.ShapeDtypeStruct((SEQ, D_MODEL), jnp.float32),
        mesh=plsc.VectorSubcoreMesh(core_axis_name="c", subcore_axis_name="s"),
        scratch_types=[
            pltpu.VMEM((CH, D_MODEL), jnp.float32),
            pltpu.VMEM((CH, D_MODEL), jnp.float32),
            pltpu.VMEM((CH,), jnp.int32),
            pltpu.VMEM((CH,), jnp.int32),
            pltpu.SemaphoreType.DMA,
            pltpu.SemaphoreType.DMA,
        ],
    )(_combine_body)


def _combine_body(y_hbm, posw_hbm, out_hbm, r0, r1, i0, i1, sem0, sem1):
    wid = lax.axis_index("s") * 2 + lax.axis_index("c")
    base = wid * TPW
    for c in range(TPW // CH):
        tb = base + c * CH
        pltpu.sync_copy(posw_hbm.at[wid, 0, pl.ds(c * CH, CH)], i0)
        pltpu.sync_copy(posw_hbm.at[wid, 1, pl.ds(c * CH, CH)], i1)
        cp0 = pltpu.async_copy(y_hbm.at[i0], r0, sem0)
        cp1 = pltpu.async_copy(y_hbm.at[i1], r1, sem1)
        cp0.wait()
        cp1.wait()

        def body(j, _):
            for u in range(D_MODEL // 16):
                sl = pl.ds(u * 16, 16)
                r0[j, sl] = r0[j, sl] + r1[j, sl]
            return 0

        lax.fori_loop(0, CH, body, 0)
        pltpu.sync_copy(r0, out_hbm.at[pl.ds(tb, CH)])


@jax.jit
def kernel(x, gate_W, gate_b, Wi, bi, Wo, bo):
    b, s, d = x.shape
    x2 = x.reshape(s, d)
    xT = x2.T
    gwT = gate_W.T
    gb2 = gate_b.reshape(NUM_EXPERTS, 1)

    p0, p1, wA, wB, sched = _router(xT, gwT, gb2)
    st, sh, se, sv = sched[0], sched[1], sched[2], sched[3]

    posw = jnp.stack([p0.reshape(NW, TPW), p1.reshape(NW, TPW)], axis=1)
    wf16 = jnp.broadcast_to(
        jnp.concatenate([wA, wB], axis=1).reshape(A_TOTAL, 1), (A_TOTAL, 128))

    xs, ws = _get_dispatch()(x2, wf16, posw)
    y = _gmm(st, sh, se, sv, xs, ws, Wi,
             bi.reshape(NUM_EXPERTS, 1, HIDDEN), Wo,
             bo.reshape(NUM_EXPERTS, 1, D_MODEL))
    out = _get_combine()(y, posw)
    return out.reshape(b, s, d)


# speedup vs baseline: 1.8832x; 1.2117x over previous
import functools

import jax
import jax.numpy as jnp
from jax import lax
from jax.experimental import pallas as pl
from jax.experimental.pallas import tpu as pltpu
from jax.experimental.pallas import tpu_sc as plsc

D_MODEL = 1024
HIDDEN = 4096
NUM_EXPERTS = 8
SEQ = 2048
A_TOTAL = 2 * SEQ

R_TILE = 256
T_MAX = A_TOTAL // R_TILE + NUM_EXPERTS - 1
N_PAD = T_MAX * R_TILE
H_CHUNK = 1024
N_H = HIDDEN // H_CHUNK
S_STEPS = T_MAX * N_H
NW = 32
TPW = SEQ // NW


def _shift_down(x, d, axis):
    pad = jnp.zeros_like(lax.slice_in_dim(x, 0, d, axis=axis))
    rest = lax.slice_in_dim(x, 0, x.shape[axis] - d, axis=axis)
    return lax.concatenate([pad, rest], dimension=axis)


def _lane_cumsum(m):
    acc = m
    d = 1
    while d < m.shape[1]:
        acc = acc + _shift_down(acc, d, 1)
        d *= 2
    return acc


def _router_kernel(xT_ref, gwT_ref, gb_ref, p0_ref, p1_ref, wA_ref, wB_ref,
                   sched_ref):
    logits = jnp.dot(gwT_ref[...], xT_ref[...],
                     preferred_element_type=jnp.float32) + gb_ref[...]
    m = jnp.max(logits, axis=0, keepdims=True)
    ex = jnp.exp(logits - m)
    probs = ex / jnp.sum(ex, axis=0, keepdims=True)

    row = lax.broadcasted_iota(jnp.int32, probs.shape, 0)
    m1 = jnp.max(probs, axis=0, keepdims=True)
    i1 = jnp.min(jnp.where(probs == m1, row, NUM_EXPERTS), axis=0,
                 keepdims=True)
    mask1 = row == i1
    probs2 = jnp.where(mask1, -1.0, probs)
    m2 = jnp.max(probs2, axis=0, keepdims=True)
    i2 = jnp.min(jnp.where(probs2 == m2, row, NUM_EXPERTS), axis=0,
                 keepdims=True)
    mask2 = row == i2

    M0 = mask1.astype(jnp.float32)
    M1 = mask2.astype(jnp.float32)
    C0 = _lane_cumsum(M0)
    C1 = _lane_cumsum(M1)
    c0 = C0[:, SEQ - 1:SEQ]
    c1 = C1[:, SEQ - 1:SEQ]
    cnt = (c0 + c1).astype(jnp.int32)

    nt = (cnt + (R_TILE - 1)) // R_TILE
    row81 = lax.broadcasted_iota(jnp.int32, (NUM_EXPERTS, 1), 0)
    acc = nt
    d = 1
    while d < NUM_EXPERTS:
        acc = acc + _shift_down(acc, d, 0)
        d *= 2
    tilebase = acc - nt
    offp = (tilebase * R_TILE).astype(jnp.float32)

    pos0 = jnp.sum(M0 * (offp + C0 - 1.0), axis=0, keepdims=True)
    pos1 = jnp.sum(M1 * (offp + c0 + C1 - 1.0), axis=0, keepdims=True)
    p0_ref[...] = pos0.astype(jnp.int32)
    p1_ref[...] = pos1.astype(jnp.int32)
    wA_ref[...] = m1
    wB_ref[...] = m2

    tb_f = tilebase.astype(jnp.float32)
    nt_f = nt.astype(jnp.float32)
    s_f = lax.broadcasted_iota(jnp.int32, (1, 128), 1).astype(jnp.float32)
    z = jnp.zeros((1, 128), jnp.float32)
    a_tile, a_hc, a_e, a_v = z, z, z, z
    a_nge, a_nghc = z, z
    nt_s = [None] * NUM_EXPERTS
    for e in range(NUM_EXPERTS):
        sel = (row81 == e).astype(jnp.float32)
        nt_s[e] = jnp.sum(sel * nt_f)
    for e in range(NUM_EXPERTS):
        sel = (row81 == e).astype(jnp.float32)
        tb_e = jnp.sum(sel * tb_f)
        nt_e = nt_s[e]
        nxt = jnp.float32(e)
        for e2 in range(NUM_EXPERTS - 1, e, -1):
            nxt = jnp.where(nt_s[e2] > 0, jnp.float32(e2), nxt)
        has_next = nxt != float(e)
        sb_e = 4.0 * tb_e
        in_e = jnp.logical_and(s_f >= sb_e, s_f < sb_e + 4.0 * nt_e)
        ds = s_f - sb_e
        q = jnp.floor((ds + 0.5) / nt_e)
        rem = ds - q * nt_e
        a_tile = a_tile + jnp.where(in_e, tb_e + rem, 0.0)
        a_hc = a_hc + jnp.where(in_e, q, 0.0)
        a_e = a_e + jnp.where(in_e, float(e), 0.0)
        a_v = a_v + jnp.where(in_e, 1.0, 0.0)
        nge_l = jnp.where(q < 3.0, float(e), nxt)
        nghc_l = jnp.where(q < 3.0, q + 1.0,
                           jnp.where(has_next, 0.0, 3.0))
        a_nge = a_nge + jnp.where(in_e, nge_l, 0.0)
        a_nghc = a_nghc + jnp.where(in_e, nghc_l, 0.0)
    sched_ref[0:1, :] = a_tile.astype(jnp.int32)
    sched_ref[1:2, :] = a_hc.astype(jnp.int32)
    sched_ref[2:3, :] = a_e.astype(jnp.int32)
    sched_ref[3:4, :] = a_v.astype(jnp.int32)
    sched_ref[4:5, :] = a_nge.astype(jnp.int32)
    sched_ref[5:6, :] = a_nghc.astype(jnp.int32)


def _router(xT, gwT, gb2):
    return pl.pallas_call(
        _router_kernel,
        out_shape=(
            jax.ShapeDtypeStruct((1, SEQ), jnp.int32),
            jax.ShapeDtypeStruct((1, SEQ), jnp.int32),
            jax.ShapeDtypeStruct((1, SEQ), jnp.float32),
            jax.ShapeDtypeStruct((1, SEQ), jnp.float32),
            jax.ShapeDtypeStruct((6, 128), jnp.int32),
        ),
    )(xT, gwT, gb2)


@functools.cache
def _get_dispatch():
    return functools.partial(
        pl.kernel,
        out_type=(
            jax.ShapeDtypeStruct((N_PAD, D_MODEL), jnp.float32),
            jax.ShapeDtypeStruct((N_PAD, 128), jnp.float32),
        ),
        mesh=plsc.VectorSubcoreMesh(core_axis_name="c", subcore_axis_name="s"),
        scratch_types=[
            pltpu.VMEM((TPW, D_MODEL), jnp.float32),
            pltpu.VMEM((TPW, 128), jnp.float32),
            pltpu.VMEM((TPW, 128), jnp.float32),
            pltpu.VMEM((TPW,), jnp.int32),
            pltpu.VMEM((TPW,), jnp.int32),
            pltpu.SemaphoreType.DMA,
        ],
    )(_dispatch_body)


def _dispatch_body(x_hbm, wf_hbm, posw_hbm, xs_hbm, ws_hbm,
                   xrows, wr0, wr1, pidx0, pidx1, sem):
    wid = lax.axis_index("s") * 2 + lax.axis_index("c")
    base = wid * TPW
    pltpu.sync_copy(posw_hbm.at[wid, 0], pidx0)
    pltpu.sync_copy(posw_hbm.at[wid, 1], pidx1)
    pltpu.sync_copy(x_hbm.at[pl.ds(base, TPW)], xrows)
    pltpu.sync_copy(wf_hbm.at[pl.ds(base, TPW)], wr0)
    pltpu.sync_copy(wf_hbm.at[pl.ds(SEQ + base, TPW)], wr1)
    pltpu.async_copy(xrows, xs_hbm.at[pidx0], sem).wait()
    pltpu.async_copy(xrows, xs_hbm.at[pidx1], sem).wait()
    pltpu.async_copy(wr0, ws_hbm.at[pidx0], sem).wait()
    pltpu.async_copy(wr1, ws_hbm.at[pidx1], sem).wait()


def _gmm_kernel(st, sh, se, sv, nge, nghc, xs_ref, ws_ref, wi_hbm, bi_ref,
                wo_hbm, bo_ref, y_ref, wi_buf, wo_buf, wi_bf, wo_bf,
                semA, semB):
    s = pl.program_id(0)
    hc = sh[s]
    rows = pl.ds(st[s] * R_TILE, R_TILE)
    prev = jnp.maximum(s - 1, 0)
    new_w = jnp.logical_or(
        s == 0, jnp.logical_or(se[s] != se[prev], sh[s] != sh[prev]))
    valid = sv[s] == 1
    slot = jnp.remainder(hc, 2)

    def wi_copy(e, h, sl):
        return pltpu.make_async_copy(
            wi_hbm.at[e, :, pl.ds(h * H_CHUNK, H_CHUNK)],
            wi_buf.at[sl], semA.at[sl])

    def wo_copy(e, h, sl):
        return pltpu.make_async_copy(
            wo_hbm.at[e, pl.ds(h * H_CHUNK, H_CHUNK), :],
            wo_buf.at[sl], semB.at[sl])

    @pl.when(jnp.logical_and(valid, s == 0))
    def _():
        wi_copy(se[s], hc, slot).start()
        wo_copy(se[s], hc, slot).start()

    @pl.when(jnp.logical_and(valid, new_w))
    def _():
        wi_copy(se[s], hc, slot).wait()
        wo_copy(se[s], hc, slot).wait()
        wi_bf[...] = wi_buf[slot].astype(jnp.bfloat16)
        wo_bf[...] = wo_buf[slot].astype(jnp.bfloat16)
        wi_copy(nge[s], nghc[s], 1 - slot).start()
        wo_copy(nge[s], nghc[s], 1 - slot).start()

    @pl.when(valid)
    def _():
        xb = xs_ref[...].astype(jnp.bfloat16)
        w_col = ws_ref[...][:, 0:1]
        h = jnp.dot(xb, wi_bf[...], preferred_element_type=jnp.float32)
        h = jnp.maximum(h + bi_ref[0], 0.0)
        part = jnp.dot(h.astype(jnp.bfloat16), wo_bf[...],
                       preferred_element_type=jnp.float32)

        @pl.when(hc == 0)
        def _():
            y_ref[rows, :] = (part + bo_ref[0]) * w_col

        @pl.when(hc != 0)
        def _():
            y_ref[rows, :] += part * w_col

    @pl.when(jnp.logical_and(s > 0,
                             jnp.logical_and(sv[s] == 0, sv[prev] == 1)))
    def _():
        wi_copy(0, 0, 0).wait()
        wo_copy(0, 0, 0).wait()


def _gmm(st, sh, se, sv, nge, nghc, xs, ws, Wi, bi3, Wo, bo3):
    grid_spec = pltpu.PrefetchScalarGridSpec(
        num_scalar_prefetch=6,
        grid=(S_STEPS + 1,),
        in_specs=[
            pl.BlockSpec((R_TILE, D_MODEL),
                         lambda s, st, sh, se, sv, ng, nh: (st[s], 0)),
            pl.BlockSpec((R_TILE, 128),
                         lambda s, st, sh, se, sv, ng, nh: (st[s], 0)),
            pl.BlockSpec(memory_space=pl.ANY),
            pl.BlockSpec((1, 1, H_CHUNK),
                         lambda s, st, sh, se, sv, ng, nh: (se[s], 0, sh[s])),
            pl.BlockSpec(memory_space=pl.ANY),
            pl.BlockSpec((1, 1, D_MODEL),
                         lambda s, st, sh, se, sv, ng, nh: (se[s], 0, 0)),
        ],
        out_specs=pl.BlockSpec((N_PAD, D_MODEL),
                               lambda s, st, sh, se, sv, ng, nh: (0, 0)),
        scratch_shapes=[
            pltpu.VMEM((2, D_MODEL, H_CHUNK), jnp.float32),
            pltpu.VMEM((2, H_CHUNK, D_MODEL), jnp.float32),
            pltpu.VMEM((D_MODEL, H_CHUNK), jnp.bfloat16),
            pltpu.VMEM((H_CHUNK, D_MODEL), jnp.bfloat16),
            pltpu.SemaphoreType.DMA((2,)),
            pltpu.SemaphoreType.DMA((2,)),
        ],
    )
    return pl.pallas_call(
        _gmm_kernel,
        grid_spec=grid_spec,
        out_shape=jax.ShapeDtypeStruct((N_PAD, D_MODEL), jnp.float32),
    )(st, sh, se, sv, nge, nghc, xs, ws, Wi, bi3, Wo, bo3)


CH = 32

@functools.cache
def _get_combine():
    return functools.partial(
        pl.kernel,
        out_type=jax.ShapeDtypeStruct((SEQ, D_MODEL), jnp.float32),
        mesh=plsc.VectorSubcoreMesh(core_axis_name="c", subcore_axis_name="s"),
        scratch_types=[
            pltpu.VMEM((CH, D_MODEL), jnp.float32),
            pltpu.VMEM((CH, D_MODEL), jnp.float32),
            pltpu.VMEM((CH,), jnp.int32),
            pltpu.VMEM((CH,), jnp.int32),
            pltpu.SemaphoreType.DMA,
            pltpu.SemaphoreType.DMA,
        ],
    )(_combine_body)


def _combine_body(y_hbm, posw_hbm, out_hbm, r0, r1, i0, i1, sem0, sem1):
    wid = lax.axis_index("s") * 2 + lax.axis_index("c")
    base = wid * TPW
    for c in range(TPW // CH):
        tb = base + c * CH
        pltpu.sync_copy(posw_hbm.at[wid, 0, pl.ds(c * CH, CH)], i0)
        pltpu.sync_copy(posw_hbm.at[wid, 1, pl.ds(c * CH, CH)], i1)
        cp0 = pltpu.async_copy(y_hbm.at[i0], r0, sem0)
        cp1 = pltpu.async_copy(y_hbm.at[i1], r1, sem1)
        cp0.wait()
        cp1.wait()

        def body(j, _):
            for u in range(D_MODEL // 16):
                sl = pl.ds(u * 16, 16)
                r0[j, sl] = r0[j, sl] + r1[j, sl]
            return 0

        lax.fori_loop(0, CH, body, 0)
        pltpu.sync_copy(r0, out_hbm.at[pl.ds(tb, CH)])


@jax.jit
def kernel(x, gate_W, gate_b, Wi, bi, Wo, bo):
    b, s, d = x.shape
    x2 = x.reshape(s, d)
    xT = x2.T
    gwT = gate_W.T
    gb2 = gate_b.reshape(NUM_EXPERTS, 1)

    p0, p1, wA, wB, sched = _router(xT, gwT, gb2)
    st, sh, se, sv = sched[0], sched[1], sched[2], sched[3]
    nge, nghc = sched[4], sched[5]

    posw = jnp.stack([p0.reshape(NW, TPW), p1.reshape(NW, TPW)], axis=1)
    wf16 = jnp.broadcast_to(
        jnp.concatenate([wA, wB], axis=1).reshape(A_TOTAL, 1), (A_TOTAL, 128))

    xs, ws = _get_dispatch()(x2, wf16, posw)
    y = _gmm(st, sh, se, sv, nge, nghc, xs, ws, Wi,
             bi.reshape(NUM_EXPERTS, 1, HIDDEN), Wo,
             bo.reshape(NUM_EXPERTS, 1, D_MODEL))
    out = _get_combine()(y, posw)
    return out.reshape(b, s, d)


# speedup vs baseline: 2.0379x; 1.0821x over previous
import functools

import jax
import jax.numpy as jnp
from jax import lax
from jax.experimental import pallas as pl
from jax.experimental.pallas import tpu as pltpu
from jax.experimental.pallas import tpu_sc as plsc

D_MODEL = 1024
HIDDEN = 4096
NUM_EXPERTS = 8
SEQ = 2048
A_TOTAL = 2 * SEQ

R_TILE = 256
T_MAX = A_TOTAL // R_TILE + NUM_EXPERTS - 1
N_PAD = T_MAX * R_TILE
H_CHUNK = 1024
N_H = HIDDEN // H_CHUNK
S_STEPS = T_MAX * N_H
NW = 32
TPW = SEQ // NW


def _shift_down(x, d, axis):
    pad = jnp.zeros_like(lax.slice_in_dim(x, 0, d, axis=axis))
    rest = lax.slice_in_dim(x, 0, x.shape[axis] - d, axis=axis)
    return lax.concatenate([pad, rest], dimension=axis)


def _lane_cumsum(m):
    acc = m
    d = 1
    while d < m.shape[1]:
        acc = acc + _shift_down(acc, d, 1)
        d *= 2
    return acc


def _router_kernel(xT_ref, gwT_ref, gb_ref, p0_ref, p1_ref, wA_ref, wB_ref,
                   sched_ref):
    logits = jnp.dot(gwT_ref[...], xT_ref[...],
                     preferred_element_type=jnp.float32) + gb_ref[...]
    m = jnp.max(logits, axis=0, keepdims=True)
    ex = jnp.exp(logits - m)
    probs = ex / jnp.sum(ex, axis=0, keepdims=True)

    row = lax.broadcasted_iota(jnp.int32, probs.shape, 0)
    m1 = jnp.max(probs, axis=0, keepdims=True)
    i1 = jnp.min(jnp.where(probs == m1, row, NUM_EXPERTS), axis=0,
                 keepdims=True)
    mask1 = row == i1
    probs2 = jnp.where(mask1, -1.0, probs)
    m2 = jnp.max(probs2, axis=0, keepdims=True)
    i2 = jnp.min(jnp.where(probs2 == m2, row, NUM_EXPERTS), axis=0,
                 keepdims=True)
    mask2 = row == i2

    M0 = mask1.astype(jnp.float32)
    M1 = mask2.astype(jnp.float32)
    C0 = _lane_cumsum(M0)
    C1 = _lane_cumsum(M1)
    c0 = C0[:, SEQ - 1:SEQ]
    c1 = C1[:, SEQ - 1:SEQ]
    cnt = (c0 + c1).astype(jnp.int32)

    nt = (cnt + (R_TILE - 1)) // R_TILE
    row81 = lax.broadcasted_iota(jnp.int32, (NUM_EXPERTS, 1), 0)
    acc = nt
    d = 1
    while d < NUM_EXPERTS:
        acc = acc + _shift_down(acc, d, 0)
        d *= 2
    tilebase = acc - nt
    offp = (tilebase * R_TILE).astype(jnp.float32)

    pos0 = jnp.sum(M0 * (offp + C0 - 1.0), axis=0, keepdims=True)
    pos1 = jnp.sum(M1 * (offp + c0 + C1 - 1.0), axis=0, keepdims=True)
    p0_ref[...] = pos0.astype(jnp.int32)
    p1_ref[...] = pos1.astype(jnp.int32)
    wA_ref[...] = m1
    wB_ref[...] = m2

    tb_f = tilebase.astype(jnp.float32)
    nt_f = nt.astype(jnp.float32)
    s_f = lax.broadcasted_iota(jnp.int32, (1, 128), 1).astype(jnp.float32)
    z = jnp.zeros((1, 128), jnp.float32)
    a_tile, a_hc, a_e, a_v = z, z, z, z
    a_nge, a_nghc = z, z
    nt_s = [None] * NUM_EXPERTS
    for e in range(NUM_EXPERTS):
        sel = (row81 == e).astype(jnp.float32)
        nt_s[e] = jnp.sum(sel * nt_f)
    for e in range(NUM_EXPERTS):
        sel = (row81 == e).astype(jnp.float32)
        tb_e = jnp.sum(sel * tb_f)
        nt_e = nt_s[e]
        nxt = jnp.float32(e)
        for e2 in range(NUM_EXPERTS - 1, e, -1):
            nxt = jnp.where(nt_s[e2] > 0, jnp.float32(e2), nxt)
        has_next = nxt != float(e)
        sb_e = 4.0 * tb_e
        in_e = jnp.logical_and(s_f >= sb_e, s_f < sb_e + 4.0 * nt_e)
        ds = s_f - sb_e
        q = jnp.floor((ds + 0.5) / nt_e)
        rem = ds - q * nt_e
        a_tile = a_tile + jnp.where(in_e, tb_e + rem, 0.0)
        a_hc = a_hc + jnp.where(in_e, q, 0.0)
        a_e = a_e + jnp.where(in_e, float(e), 0.0)
        a_v = a_v + jnp.where(in_e, 1.0, 0.0)
        nge_l = jnp.where(q < 3.0, float(e), nxt)
        nghc_l = jnp.where(q < 3.0, q + 1.0,
                           jnp.where(has_next, 0.0, 3.0))
        a_nge = a_nge + jnp.where(in_e, nge_l, 0.0)
        a_nghc = a_nghc + jnp.where(in_e, nghc_l, 0.0)
    sched_ref[0:1, :] = a_tile.astype(jnp.int32)
    sched_ref[1:2, :] = a_hc.astype(jnp.int32)
    sched_ref[2:3, :] = a_e.astype(jnp.int32)
    sched_ref[3:4, :] = a_v.astype(jnp.int32)
    sched_ref[4:5, :] = a_nge.astype(jnp.int32)
    sched_ref[5:6, :] = a_nghc.astype(jnp.int32)


def _router(xT, gwT, gb2):
    return pl.pallas_call(
        _router_kernel,
        out_shape=(
            jax.ShapeDtypeStruct((1, SEQ), jnp.int32),
            jax.ShapeDtypeStruct((1, SEQ), jnp.int32),
            jax.ShapeDtypeStruct((1, SEQ), jnp.float32),
            jax.ShapeDtypeStruct((1, SEQ), jnp.float32),
            jax.ShapeDtypeStruct((6, 128), jnp.int32),
        ),
    )(xT, gwT, gb2)


@functools.cache
def _get_dispatch():
    return functools.partial(
        pl.kernel,
        out_type=(
            jax.ShapeDtypeStruct((N_PAD, D_MODEL), jnp.float32),
            jax.ShapeDtypeStruct((N_PAD, 128), jnp.float32),
        ),
        mesh=plsc.VectorSubcoreMesh(core_axis_name="c", subcore_axis_name="s"),
        scratch_types=[
            pltpu.VMEM((TPW, D_MODEL), jnp.float32),
            pltpu.VMEM((TPW, 128), jnp.float32),
            pltpu.VMEM((TPW, 128), jnp.float32),
            pltpu.VMEM((TPW,), jnp.int32),
            pltpu.VMEM((TPW,), jnp.int32),
            pltpu.SemaphoreType.DMA,
        ],
    )(_dispatch_body)


def _dispatch_body(x_hbm, wf_hbm, posw_hbm, xs_hbm, ws_hbm,
                   xrows, wr0, wr1, pidx0, pidx1, sem):
    wid = lax.axis_index("s") * 2 + lax.axis_index("c")
    base = wid * TPW
    pltpu.sync_copy(posw_hbm.at[wid, 0], pidx0)
    pltpu.sync_copy(posw_hbm.at[wid, 1], pidx1)
    pltpu.sync_copy(x_hbm.at[pl.ds(base, TPW)], xrows)
    pltpu.sync_copy(wf_hbm.at[pl.ds(base, TPW)], wr0)
    pltpu.sync_copy(wf_hbm.at[pl.ds(SEQ + base, TPW)], wr1)
    pltpu.async_copy(xrows, xs_hbm.at[pidx0], sem).wait()
    pltpu.async_copy(xrows, xs_hbm.at[pidx1], sem).wait()
    pltpu.async_copy(wr0, ws_hbm.at[pidx0], sem).wait()
    pltpu.async_copy(wr1, ws_hbm.at[pidx1], sem).wait()


def _gmm_kernel(st, sh, se, sv, nge, nghc, xs_ref, ws_ref, wi_hbm, bi_ref,
                wo_hbm, bo_ref, y_ref, xbf, wi_buf, wo_buf,
                semA, semB):
    s = pl.program_id(0)
    hc = sh[s]
    rows = pl.ds(st[s] * R_TILE, R_TILE)
    prev = jnp.maximum(s - 1, 0)
    new_w = jnp.logical_or(
        s == 0, jnp.logical_or(se[s] != se[prev], sh[s] != sh[prev]))
    valid = sv[s] == 1
    slot = jnp.remainder(hc, 2)

    def wi_copy(e, h, sl):
        return pltpu.make_async_copy(
            wi_hbm.at[e, :, pl.ds(h * H_CHUNK, H_CHUNK)],
            wi_buf.at[sl], semA.at[sl])

    def wo_copy(e, h, sl):
        return pltpu.make_async_copy(
            wo_hbm.at[e, pl.ds(h * H_CHUNK, H_CHUNK), :],
            wo_buf.at[sl], semB.at[sl])

    @pl.when(jnp.logical_and(valid, s == 0))
    def _():
        wi_copy(se[s], hc, slot).start()
        wo_copy(se[s], hc, slot).start()

    @pl.when(jnp.logical_and(valid, new_w))
    def _():
        wi_copy(se[s], hc, slot).wait()
        wo_copy(se[s], hc, slot).wait()
        wi_copy(nge[s], nghc[s], 1 - slot).start()
        wo_copy(nge[s], nghc[s], 1 - slot).start()

    @pl.when(jnp.logical_and(valid, hc == 0))
    def _():
        xbf[rows, :] = xs_ref[...].astype(jnp.bfloat16)

    @pl.when(valid)
    def _():
        xb = xbf[rows, :]
        w_col = ws_ref[...][:, 0:1]
        h = jnp.dot(xb, wi_buf[slot].astype(jnp.bfloat16),
                    preferred_element_type=jnp.float32)
        h = jnp.maximum(h + bi_ref[0], 0.0)
        part = jnp.dot(h.astype(jnp.bfloat16),
                       wo_buf[slot].astype(jnp.bfloat16),
                       preferred_element_type=jnp.float32)

        @pl.when(hc == 0)
        def _():
            y_ref[rows, :] = (part + bo_ref[0]) * w_col

        @pl.when(hc != 0)
        def _():
            y_ref[rows, :] += part * w_col

    @pl.when(jnp.logical_and(s > 0,
                             jnp.logical_and(sv[s] == 0, sv[prev] == 1)))
    def _():
        wi_copy(0, 0, 0).wait()
        wo_copy(0, 0, 0).wait()


def _gmm(st, sh, se, sv, nge, nghc, xs, ws, Wi, bi3, Wo, bo3):
    grid_spec = pltpu.PrefetchScalarGridSpec(
        num_scalar_prefetch=6,
        grid=(S_STEPS + 1,),
        in_specs=[
            pl.BlockSpec((R_TILE, D_MODEL),
                         lambda s, st, sh, se, sv, ng, nh: (
                             jnp.where(sh[s] == 0, st[s], 0), 0)),
            pl.BlockSpec((R_TILE, 128),
                         lambda s, st, sh, se, sv, ng, nh: (st[s], 0)),
            pl.BlockSpec(memory_space=pl.ANY),
            pl.BlockSpec((1, 1, H_CHUNK),
                         lambda s, st, sh, se, sv, ng, nh: (se[s], 0, sh[s])),
            pl.BlockSpec(memory_space=pl.ANY),
            pl.BlockSpec((1, 1, D_MODEL),
                         lambda s, st, sh, se, sv, ng, nh: (se[s], 0, 0)),
        ],
        out_specs=pl.BlockSpec((N_PAD, D_MODEL),
                               lambda s, st, sh, se, sv, ng, nh: (0, 0)),
        scratch_shapes=[
            pltpu.VMEM((N_PAD, D_MODEL), jnp.bfloat16),
            pltpu.VMEM((2, D_MODEL, H_CHUNK), jnp.float32),
            pltpu.VMEM((2, H_CHUNK, D_MODEL), jnp.float32),
            pltpu.SemaphoreType.DMA((2,)),
            pltpu.SemaphoreType.DMA((2,)),
        ],
    )
    return pl.pallas_call(
        _gmm_kernel,
        grid_spec=grid_spec,
        out_shape=jax.ShapeDtypeStruct((N_PAD, D_MODEL), jnp.float32),
    )(st, sh, se, sv, nge, nghc, xs, ws, Wi, bi3, Wo, bo3)


CH = 32

@functools.cache
def _get_combine():
    return functools.partial(
        pl.kernel,
        out_type=jax.ShapeDtypeStruct((SEQ, D_MODEL), jnp.float32),
        mesh=plsc.VectorSubcoreMesh(core_axis_name="c", subcore_axis_name="s"),
        scratch_types=[
            pltpu.VMEM((CH, D_MODEL), jnp.float32),
            pltpu.VMEM((CH, D_MODEL), jnp.float32),
            pltpu.VMEM((CH,), jnp.int32),
            pltpu.VMEM((CH,), jnp.int32),
            pltpu.SemaphoreType.DMA,
            pltpu.SemaphoreType.DMA,
        ],
    )(_combine_body)


def _combine_body(y_hbm, posw_hbm, out_hbm, r0, r1, i0, i1, sem0, sem1):
    wid = lax.axis_index("s") * 2 + lax.axis_index("c")
    base = wid * TPW
    for c in range(TPW // CH):
        tb = base + c * CH
        pltpu.sync_copy(posw_hbm.at[wid, 0, pl.ds(c * CH, CH)], i0)
        pltpu.sync_copy(posw_hbm.at[wid, 1, pl.ds(c * CH, CH)], i1)
        cp0 = pltpu.async_copy(y_hbm.at[i0], r0, sem0)
        cp1 = pltpu.async_copy(y_hbm.at[i1], r1, sem1)
        cp0.wait()
        cp1.wait()

        def body(j, _):
            for u in range(D_MODEL // 16):
                sl = pl.ds(u * 16, 16)
                r0[j, sl] = r0[j, sl] + r1[j, sl]
            return 0

        lax.fori_loop(0, CH, body, 0)
        pltpu.sync_copy(r0, out_hbm.at[pl.ds(tb, CH)])


@jax.jit
def kernel(x, gate_W, gate_b, Wi, bi, Wo, bo):
    b, s, d = x.shape
    x2 = x.reshape(s, d)
    xT = x2.T
    gwT = gate_W.T
    gb2 = gate_b.reshape(NUM_EXPERTS, 1)

    p0, p1, wA, wB, sched = _router(xT, gwT, gb2)
    st, sh, se, sv = sched[0], sched[1], sched[2], sched[3]
    nge, nghc = sched[4], sched[5]

    posw = jnp.stack([p0.reshape(NW, TPW), p1.reshape(NW, TPW)], axis=1)
    wf16 = jnp.broadcast_to(
        jnp.concatenate([wA, wB], axis=1).reshape(A_TOTAL, 1), (A_TOTAL, 128))

    xs, ws = _get_dispatch()(x2, wf16, posw)
    y = _gmm(st, sh, se, sv, nge, nghc, xs, ws, Wi,
             bi.reshape(NUM_EXPERTS, 1, HIDDEN), Wo,
             bo.reshape(NUM_EXPERTS, 1, D_MODEL))
    out = _get_combine()(y, posw)
    return out.reshape(b, s, d)
